# baseline (speedup 1.0000x reference)
import functools

import jax
import jax.numpy as jnp
from jax import lax
from jax.experimental import pallas as pl
from jax.experimental.pallas import tpu as pltpu
from jax.experimental.pallas import tpu_sc as plsc

N_ROWS = 100001
D = 128
B_TOTAL = 1024 * 200
NUM_WORKERS = 32
B_PER_W = B_TOTAL // NUM_WORKERS
CHUNK = 200
N_CHUNKS = B_PER_W // CHUNK
NBUF = 4

_mesh = plsc.VectorSubcoreMesh(core_axis_name="c", subcore_axis_name="s")


@functools.partial(
    pl.kernel,
    mesh=_mesh,
    out_type=jax.ShapeDtypeStruct((B_TOTAL, D), jnp.float32),
    scratch_types=(
        [pltpu.VMEM((B_PER_W,), jnp.int32)]
        + [pltpu.VMEM((CHUNK, D), jnp.float32) for _ in range(NBUF)]
        + [pltpu.SemaphoreType.DMA for _ in range(2 * NBUF)]
    ),
)
def _gather_kernel(idx_hbm, table_hbm, out_hbm, idx_all, *bufs):
    rows = bufs[:NBUF]
    gsem = bufs[NBUF:2 * NBUF]
    ssem = bufs[2 * NBUF:]
    wid = lax.axis_index("s") * 2 + lax.axis_index("c")
    base = wid * B_PER_W

    pltpu.sync_copy(idx_hbm.at[pl.ds(base, B_PER_W)], idx_all)

    def gather_start(g, b):
        pltpu.async_copy(
            table_hbm.at[idx_all.at[pl.ds(g * CHUNK, CHUNK)]], rows[b], gsem[b])

    def gather_wait(g, b):
        pltpu.make_async_copy(
            table_hbm.at[idx_all.at[pl.ds(g * CHUNK, CHUNK)]], rows[b], gsem[b]).wait()

    def store_start(g, b):
        pltpu.async_copy(rows[b], out_hbm.at[pl.ds(base + g * CHUNK, CHUNK)], ssem[b])

    def store_wait(g, b):
        pltpu.make_async_copy(
            rows[b], out_hbm.at[pl.ds(base + g * CHUNK, CHUNK)], ssem[b]).wait()

    for b in range(NBUF):
        gather_start(b, b)

    def outer(go, carry):
        for b in range(NBUF):
            g = NBUF * go + b
            gather_wait(g, b)
            store_start(g, b)
            store_wait(g, b)
            gather_start(g + NBUF, b)
        return carry

    lax.fori_loop(0, N_CHUNKS // NBUF - 1, outer, 0)

    for b in range(NBUF):
        g = N_CHUNKS - NBUF + b
        gather_wait(g, b)
        store_start(g, b)
    for b in range(NBUF):
        g = N_CHUNKS - NBUF + b
        store_wait(g, b)


def kernel(batch, emb_table):
    idx = batch.reshape(-1)
    out = _gather_kernel(idx, emb_table)
    return out.reshape(batch.shape[0], batch.shape[1], D)
